# baseline (speedup 1.0000x reference)
import math

import jax
import jax.numpy as jnp
from jax.experimental import pallas as pl
from jax.experimental.pallas import tpu as pltpu

_CH = 2048
_NBUF = 4
_NEG = -1e30


def _attn_kernel(starts_ref, ends_ref, enc_ref, wqt_ref, bq_ref, wkt_ref,
                 bk_ref, wvt_ref, bv_ref, soc_hbm, out_ref, bufs, sems):
    b, d = out_ref.shape
    t = soc_hbm.shape[0]
    nch = t // _CH

    def copy(i):
        slot = i % _NBUF
        return pltpu.make_async_copy(
            soc_hbm.at[pl.ds(i * _CH, _CH), :], bufs.at[slot], sems.at[slot])

    for i in range(min(_NBUF, nch)):
        copy(i).start()

    q = jnp.dot(enc_ref[...], wqt_ref[...],
                preferred_element_type=jnp.float32) + bq_ref[...]
    q = jnp.maximum(q, 0.0) * (1.0 / math.sqrt(d))

    starts = starts_ref[...]
    ends = ends_ref[...]
    wkt, bk = wkt_ref[...], bk_ref[...]
    wvt, bv = wvt_ref[...], bv_ref[...]

    m = jnp.full((b, 1), _NEG, jnp.float32)
    s = jnp.zeros((b, 1), jnp.float32)
    acc = jnp.zeros((b, d), jnp.float32)

    for j in range(nch):
        copy(j).wait()
        tok = bufs[j % _NBUF]
        k = jnp.maximum(jnp.dot(tok, wkt,
                                preferred_element_type=jnp.float32) + bk, 0.0)
        v = jnp.maximum(jnp.dot(tok, wvt,
                                preferred_element_type=jnp.float32) + bv, 0.0)

        logits = jax.lax.dot_general(
            q, k, (((1,), (1,)), ((), ())),
            preferred_element_type=jnp.float32)
        col = j * _CH + jax.lax.broadcasted_iota(jnp.int32, (b, _CH), 1)
        mask = (col >= starts) & (col < ends)
        logits = jnp.where(mask, logits, _NEG)

        m_new = jnp.maximum(m, jnp.max(logits, axis=1, keepdims=True))
        alpha = jnp.exp(m - m_new)
        p = jnp.exp(logits - m_new)
        s = s * alpha + jnp.sum(p, axis=1, keepdims=True)
        acc = acc * alpha + jnp.dot(p, v, preferred_element_type=jnp.float32)
        m = m_new

        if j + _NBUF < nch:
            copy(j + _NBUF).start()

    out_ref[...] = acc / s


def kernel(enc_hidden, social_ht, neighbors_idx_start, neighbors_idx_end,
           Wq, bq, Wk, bk, Wv, bv):
    b, d = enc_hidden.shape

    starts = neighbors_idx_start.astype(jnp.int32).reshape(b, 1)
    ends = neighbors_idx_end.astype(jnp.int32).reshape(b, 1)

    vmem = pl.BlockSpec(memory_space=pltpu.MemorySpace.VMEM)
    out = pl.pallas_call(
        _attn_kernel,
        in_specs=[vmem, vmem, vmem, vmem, vmem, vmem, vmem, vmem, vmem,
                  pl.BlockSpec(memory_space=pltpu.MemorySpace.HBM)],
        out_specs=vmem,
        out_shape=jax.ShapeDtypeStruct((b, d), jnp.float32),
        scratch_shapes=[
            pltpu.VMEM((_NBUF, _CH, d), jnp.float32),
            pltpu.SemaphoreType.DMA((_NBUF,)),
        ],
    )(starts, ends, enc_hidden,
      Wq.T, bq.reshape(1, d),
      Wk.T, bk.reshape(1, d),
      Wv.T, bv.reshape(1, d), social_ht)
    return out
